# Initial kernel scaffold; baseline (speedup 1.0000x reference)
import jax
import jax.numpy as jnp
from jax.experimental import pallas as pl


def kernel(x, codebook):
    raise NotImplementedError("write your pallas kernel here")


# speedup vs baseline: 5.9400x; 5.9400x over previous
import jax
import jax.numpy as jnp
from jax.experimental import pallas as pl

_K = 1024
_C = 32


def _vq_body(x_ref, cb_ref, out_ref):
    xb = x_ref[0]
    cb = cb_ref[...]
    cbn = jnp.sum(cb * cb, axis=1, keepdims=True)
    s = jax.lax.dot_general(
        cb, xb, (((1,), (0,)), ((), ())),
        preferred_element_type=jnp.float32,
        precision=jax.lax.Precision.HIGHEST)
    scores = cbn - 2.0 * s
    m = jnp.min(scores, axis=0, keepdims=True)
    iota_k = jax.lax.broadcasted_iota(jnp.int32, scores.shape, 0)
    idx = jnp.min(jnp.where(scores == m, iota_k, _K), axis=0)
    onehot = (iota_k == idx[None, :]).astype(jnp.float32)
    y = jax.lax.dot_general(
        cb, onehot, (((0,), (0,)), ((), ())),
        preferred_element_type=jnp.float32,
        precision=jax.lax.Precision.HIGHEST)
    out_ref[0] = y


def kernel(x, codebook):
    N, C, H, W = x.shape
    hw = H * W
    xr = x.reshape(N, C, hw)
    y = pl.pallas_call(
        _vq_body,
        grid=(N,),
        in_specs=[
            pl.BlockSpec((1, C, hw), lambda i: (i, 0, 0)),
            pl.BlockSpec((_K, _C), lambda i: (0, 0)),
        ],
        out_specs=pl.BlockSpec((1, C, hw), lambda i: (i, 0, 0)),
        out_shape=jax.ShapeDtypeStruct((N, C, hw), jnp.float32),
    )(xr, codebook)
    y = y.reshape(N, C, H, W)
    return (y, y)
